# Initial kernel scaffold; baseline (speedup 1.0000x reference)
import jax
import jax.numpy as jnp
from jax.experimental import pallas as pl


def kernel(x, edge_index, edge_attr, W1, b1, W2, b2, LW1, Lb1, LW2, Lb2):
    raise NotImplementedError("write your pallas kernel here")


# speedup vs baseline: 9.5916x; 9.5916x over previous
import functools

import jax
import jax.numpy as jnp
from jax import lax
from jax.experimental import pallas as pl
from jax.experimental.pallas import tpu as pltpu
from jax.experimental.pallas import tpu_sc as plsc

N_NODES = 10000
N_EDGES = 320000
NC = 2
NS = 16
NW = NC * NS
C = 80
EPW = N_EDGES // NW
RPW = EPW // C
NP = 10240
NPS = NP // NS
H = 64

_mesh = plsc.VectorSubcoreMesh(
    core_axis_name="c", subcore_axis_name="s", num_cores=NC, num_subcores=NS)
_sc_params = pltpu.CompilerParams(use_tc_tiling_on_sc=False)


@functools.partial(
    pl.kernel,
    out_type=jax.ShapeDtypeStruct((NC * NP,), jnp.float32),
    mesh=_mesh,
    compiler_params=_sc_params,
    scratch_types=[
        pltpu.VMEM((RPW, C), jnp.int32),
        pltpu.VMEM((C,), jnp.float32),
        pltpu.VMEM_SHARED((NP,), jnp.float32),
    ],
)
def _deg_kernel(col3d, ones, zerosn, degp, colv, onesv, acc):
    c = lax.axis_index("c")
    s = lax.axis_index("s")
    w = c * NS + s
    pltpu.sync_copy(zerosn.at[pl.ds(s * NPS, NPS)], acc.at[pl.ds(s * NPS, NPS)])
    pltpu.sync_copy(ones, onesv)
    pltpu.sync_copy(col3d.at[w], colv)
    plsc.subcore_barrier()

    def body(j, _):
        pltpu.sync_copy(onesv, acc.at[colv.at[j]], add=True)
        return 0

    lax.fori_loop(0, RPW, body, 0)
    plsc.subcore_barrier()
    pltpu.sync_copy(acc.at[pl.ds(s * NPS, NPS)],
                    degp.at[pl.ds(c * NP + s * NPS, NPS)])


@functools.partial(
    pl.kernel,
    out_type=jax.ShapeDtypeStruct((NC, NP, H), jnp.float32),
    mesh=_mesh,
    compiler_params=_sc_params,
    scratch_types=[
        pltpu.VMEM((RPW, C), jnp.int32),
        pltpu.VMEM((RPW, C), jnp.int32),
        pltpu.VMEM((C, H), jnp.float32),
        pltpu.VMEM_SHARED((NP, H), jnp.float32),
        pltpu.SemaphoreType.DMA,
    ],
)
def _seg_kernel(y, row3d, col3d, zerosnh, tp, rowv, colv, gbuf, acc, sem):
    c = lax.axis_index("c")
    s = lax.axis_index("s")
    w = c * NS + s
    pltpu.sync_copy(zerosnh.at[pl.ds(s * NPS, NPS)], acc.at[pl.ds(s * NPS, NPS)])
    pltpu.sync_copy(row3d.at[w], rowv)
    pltpu.sync_copy(col3d.at[w], colv)
    plsc.subcore_barrier()

    def body(j, _):
        pltpu.async_copy(y.at[rowv.at[j]], gbuf, sem).wait()
        pltpu.sync_copy(gbuf, acc.at[colv.at[j]], add=True)
        return 0

    lax.fori_loop(0, RPW, body, 0)
    plsc.subcore_barrier()
    pltpu.sync_copy(acc.at[pl.ds(s * NPS, NPS)], tp.at[c, pl.ds(s * NPS, NPS)])


@functools.partial(
    pl.kernel,
    out_type=(
        jax.ShapeDtypeStruct((N_EDGES, H), jnp.float32),
        jax.ShapeDtypeStruct((N_EDGES, H), jnp.float32),
    ),
    mesh=_mesh,
    compiler_params=_sc_params,
    scratch_types=[
        pltpu.VMEM((RPW, C), jnp.int32),
        pltpu.VMEM((RPW, C), jnp.int32),
        pltpu.VMEM((C, H), jnp.float32),
        pltpu.VMEM((C, H), jnp.float32),
        pltpu.SemaphoreType.DMA,
        pltpu.SemaphoreType.DMA,
    ],
)
def _edge_gather_kernel(a, b, row3d, col3d, ga, gb, rowv, colv, bufa, bufb,
                        sema, semb):
    c = lax.axis_index("c")
    s = lax.axis_index("s")
    w = c * NS + s
    pltpu.sync_copy(row3d.at[w], rowv)
    pltpu.sync_copy(col3d.at[w], colv)

    def body(j, _):
        da = pltpu.async_copy(a.at[rowv.at[j]], bufa, sema)
        db = pltpu.async_copy(b.at[colv.at[j]], bufb, semb)
        da.wait()
        db.wait()
        base = w * EPW + j * C
        pltpu.sync_copy(bufa, ga.at[pl.ds(base, C)])
        pltpu.sync_copy(bufb, gb.at[pl.ds(base, C)])
        return 0

    lax.fori_loop(0, RPW, body, 0)


BM = 1000


def _tc1_body(x_ref, w1_ref, degp_ref, y1_ref, dis_ref):
    deg = degp_ref[:, 0] + degp_ref[:, 1] + 1.0
    dis = lax.rsqrt(deg)
    xw = jnp.dot(x_ref[...], w1_ref[...], preferred_element_type=jnp.float32)
    y1_ref[...] = xw * dis[:, None]
    dis_ref[...] = dis[:, None]


def _tc1(x, w1, degp_t):
    return pl.pallas_call(
        _tc1_body,
        grid=(N_NODES // BM,),
        in_specs=[
            pl.BlockSpec((BM, 128), lambda i: (i, 0)),
            pl.BlockSpec((128, H), lambda i: (0, 0)),
            pl.BlockSpec((BM, NC), lambda i: (i, 0)),
        ],
        out_specs=[
            pl.BlockSpec((BM, H), lambda i: (i, 0)),
            pl.BlockSpec((BM, 1), lambda i: (i, 0)),
        ],
        out_shape=[
            jax.ShapeDtypeStruct((NP, H), jnp.float32),
            jax.ShapeDtypeStruct((NP, 1), jnp.float32),
        ],
    )(x, w1, degp_t)


def _tc2_body(tp_ref, y_ref, dis_ref, w_ref, b_ref, out_ref):
    dis = dis_ref[...]
    h = jnp.maximum(
        (tp_ref[0] + tp_ref[1] + y_ref[...]) * dis + b_ref[...], 0.0)
    out_ref[...] = jnp.dot(
        h, w_ref[...], preferred_element_type=jnp.float32) * dis


def _tc2(tp, y, dis, w, b):
    return pl.pallas_call(
        _tc2_body,
        grid=(N_NODES // BM,),
        in_specs=[
            pl.BlockSpec((NC, BM, H), lambda i: (0, i, 0)),
            pl.BlockSpec((BM, H), lambda i: (i, 0)),
            pl.BlockSpec((BM, 1), lambda i: (i, 0)),
            pl.BlockSpec((H, H), lambda i: (0, 0)),
            pl.BlockSpec((1, H), lambda i: (0, 0)),
        ],
        out_specs=pl.BlockSpec((BM, H), lambda i: (i, 0)),
        out_shape=jax.ShapeDtypeStruct((NP, H), jnp.float32),
    )(tp, y, dis, w, b)


def _tc3_body(tp_ref, y_ref, dis_ref, b_ref, lwa_ref, lwb_ref, lb1_ref,
              a_ref, bo_ref):
    dis = dis_ref[...]
    h = jnp.maximum(
        (tp_ref[0] + tp_ref[1] + y_ref[...]) * dis + b_ref[...], 0.0)
    a_ref[...] = jnp.dot(
        h, lwa_ref[...], preferred_element_type=jnp.float32) + lb1_ref[...]
    bo_ref[...] = jnp.dot(h, lwb_ref[...], preferred_element_type=jnp.float32)


def _tc3(tp, y, dis, b, lwa, lwb, lb1):
    return pl.pallas_call(
        _tc3_body,
        grid=(N_NODES // BM,),
        in_specs=[
            pl.BlockSpec((NC, BM, H), lambda i: (0, i, 0)),
            pl.BlockSpec((BM, H), lambda i: (i, 0)),
            pl.BlockSpec((BM, 1), lambda i: (i, 0)),
            pl.BlockSpec((1, H), lambda i: (0, 0)),
            pl.BlockSpec((H, H), lambda i: (0, 0)),
            pl.BlockSpec((H, H), lambda i: (0, 0)),
            pl.BlockSpec((1, H), lambda i: (0, 0)),
        ],
        out_specs=[
            pl.BlockSpec((BM, H), lambda i: (i, 0)),
            pl.BlockSpec((BM, H), lambda i: (i, 0)),
        ],
        out_shape=[
            jax.ShapeDtypeStruct((NP, H), jnp.float32),
            jax.ShapeDtypeStruct((NP, H), jnp.float32),
        ],
    )(tp, y, dis, b, lwa, lwb, lb1)


BE = 2000


def _tc4_body(ga_ref, gb_ref, attr_ref, wrow_ref, lw2_ref, lb2_ref, out_ref):
    z = jnp.maximum(
        ga_ref[...] + gb_ref[...] + attr_ref[...] * wrow_ref[...], 0.0)
    out_ref[...] = (
        jnp.sum(z * lw2_ref[...], axis=1, keepdims=True) + lb2_ref[...])


def _tc4(ga, gb, attr, wrow, lw2row, lb2):
    return pl.pallas_call(
        _tc4_body,
        grid=(N_EDGES // BE,),
        in_specs=[
            pl.BlockSpec((BE, H), lambda i: (i, 0)),
            pl.BlockSpec((BE, H), lambda i: (i, 0)),
            pl.BlockSpec((BE, 1), lambda i: (i, 0)),
            pl.BlockSpec((1, H), lambda i: (0, 0)),
            pl.BlockSpec((1, H), lambda i: (0, 0)),
            pl.BlockSpec((1, 1), lambda i: (0, 0)),
        ],
        out_specs=pl.BlockSpec((BE, 1), lambda i: (i, 0)),
        out_shape=jax.ShapeDtypeStruct((N_EDGES, 1), jnp.float32),
    )(ga, gb, attr, wrow, lw2row, lb2)


def kernel(x, edge_index, edge_attr, W1, b1, W2, b2, LW1, Lb1, LW2, Lb2):
    row3d = edge_index[0].astype(jnp.int32).reshape(NW, RPW, C)
    col3d = edge_index[1].astype(jnp.int32).reshape(NW, RPW, C)
    ones_c = jnp.ones((C,), jnp.float32)
    zeros_n = jnp.zeros((NP,), jnp.float32)
    zeros_nh = jnp.zeros((NP, H), jnp.float32)

    degp = _deg_kernel(col3d, ones_c, zeros_n)
    degp_t = degp.reshape(NC, NP).T

    y1, dis = _tc1(x, W1, degp_t)
    t1p = _seg_kernel(y1, row3d, col3d, zeros_nh)
    y2 = _tc2(t1p, y1, dis, W2, b1.reshape(1, H))
    t2p = _seg_kernel(y2, row3d, col3d, zeros_nh)
    a_tab, b_tab = _tc3(t2p, y2, dis, b2.reshape(1, H),
                        LW1[:H], LW1[H:2 * H], Lb1.reshape(1, H))
    ga, gb = _edge_gather_kernel(a_tab, b_tab, row3d, col3d)
    out = _tc4(ga, gb, edge_attr, LW1[2 * H].reshape(1, H),
               LW2.reshape(1, H), Lb2.reshape(1, 1))
    return out[:, 0]


# speedup vs baseline: 11.8737x; 1.2379x over previous
import functools

import jax
import jax.numpy as jnp
from jax import lax
from jax.experimental import pallas as pl
from jax.experimental.pallas import tpu as pltpu
from jax.experimental.pallas import tpu_sc as plsc

N_NODES = 10000
N_EDGES = 320000
NC = 2
NS = 16
NW = NC * NS
C = 80
EPW = N_EDGES // NW
RPW = EPW // C
NP = 10240
NPS = NP // NS
H = 64

_mesh = plsc.VectorSubcoreMesh(
    core_axis_name="c", subcore_axis_name="s", num_cores=NC, num_subcores=NS)
_sc_params = pltpu.CompilerParams(use_tc_tiling_on_sc=False)
_sc_params_nl = pltpu.CompilerParams(
    use_tc_tiling_on_sc=False, needs_layout_passes=False)


@functools.partial(
    pl.kernel,
    out_type=jax.ShapeDtypeStruct((NC * NP,), jnp.float32),
    mesh=_mesh,
    compiler_params=_sc_params,
    scratch_types=[
        pltpu.VMEM((RPW, C), jnp.int32),
        pltpu.VMEM((C,), jnp.float32),
        pltpu.VMEM_SHARED((NP,), jnp.float32),
    ],
)
def _deg_kernel(col3d, ones, zerosn, degp, colv, onesv, acc):
    c = lax.axis_index("c")
    s = lax.axis_index("s")
    w = c * NS + s
    pltpu.sync_copy(zerosn.at[pl.ds(s * NPS, NPS)], acc.at[pl.ds(s * NPS, NPS)])
    pltpu.sync_copy(ones, onesv)
    pltpu.sync_copy(col3d.at[w], colv)
    plsc.subcore_barrier()

    def body(j, _):
        pltpu.sync_copy(onesv, acc.at[colv.at[j]], add=True)
        return 0

    lax.fori_loop(0, RPW, body, 0)
    plsc.subcore_barrier()
    pltpu.sync_copy(acc.at[pl.ds(s * NPS, NPS)],
                    degp.at[pl.ds(c * NP + s * NPS, NPS)])


@functools.partial(
    pl.kernel,
    out_type=jax.ShapeDtypeStruct((NC, NP, H), jnp.float32),
    mesh=_mesh,
    compiler_params=_sc_params,
    scratch_types=[
        pltpu.VMEM((RPW, C), jnp.int32),
        pltpu.VMEM((RPW, C), jnp.int32),
        pltpu.VMEM((C, H), jnp.float32),
        pltpu.VMEM((C, H), jnp.float32),
        pltpu.VMEM_SHARED((NP, H), jnp.float32),
        pltpu.SemaphoreType.DMA,
        pltpu.SemaphoreType.DMA,
    ],
)
def _seg_kernel(y, row3d, col3d, zerosnh, tp, rowv, colv, g0, g1, acc, s0, s1):
    c = lax.axis_index("c")
    s = lax.axis_index("s")
    w = c * NS + s
    pltpu.sync_copy(zerosnh.at[pl.ds(s * NPS, NPS)], acc.at[pl.ds(s * NPS, NPS)])
    pltpu.sync_copy(row3d.at[w], rowv)
    pltpu.sync_copy(col3d.at[w], colv)
    plsc.subcore_barrier()

    pltpu.async_copy(y.at[rowv.at[0]], g0, s0)

    def pair(i, _):
        j0 = 2 * i
        pltpu.async_copy(y.at[rowv.at[j0 + 1]], g1, s1)
        pltpu.make_async_copy(y.at[rowv.at[j0]], g0, s0).wait()
        pltpu.sync_copy(g0, acc.at[colv.at[j0]], add=True)
        pltpu.async_copy(y.at[rowv.at[j0 + 2]], g0, s0)
        pltpu.make_async_copy(y.at[rowv.at[j0 + 1]], g1, s1).wait()
        pltpu.sync_copy(g1, acc.at[colv.at[j0 + 1]], add=True)
        return 0

    lax.fori_loop(0, (RPW - 1) // 2, pair, 0)
    pltpu.make_async_copy(y.at[rowv.at[RPW - 1]], g0, s0).wait()
    pltpu.sync_copy(g0, acc.at[colv.at[RPW - 1]], add=True)
    plsc.subcore_barrier()
    pltpu.sync_copy(acc.at[pl.ds(s * NPS, NPS)], tp.at[c, pl.ds(s * NPS, NPS)])


@functools.partial(
    pl.kernel,
    out_type=jax.ShapeDtypeStruct((N_EDGES,), jnp.float32),
    mesh=_mesh,
    compiler_params=_sc_params_nl,
    scratch_types=[
        pltpu.VMEM((RPW, C), jnp.int32),
        pltpu.VMEM((RPW, C), jnp.int32),
        pltpu.VMEM((C, H), jnp.float32),
        pltpu.VMEM((C, H), jnp.float32),
        pltpu.VMEM((C,), jnp.float32),
        pltpu.VMEM((C, H), jnp.float32),
        pltpu.VMEM((C, H), jnp.float32),
        pltpu.VMEM((C,), jnp.float32),
        pltpu.VMEM((H,), jnp.float32),
        pltpu.VMEM((H,), jnp.float32),
        pltpu.VMEM((16,), jnp.float32),
        pltpu.VMEM((EPW,), jnp.float32),
        pltpu.SemaphoreType.DMA,
        pltpu.SemaphoreType.DMA,
        pltpu.SemaphoreType.DMA,
        pltpu.SemaphoreType.DMA,
        pltpu.SemaphoreType.DMA,
        pltpu.SemaphoreType.DMA,
    ],
)
def _edge_mlp_kernel(a, b, row3d, col3d, attr, wrow, lw2, lb2, out,
                     rowv, colv, a0, b0, t0, a1, b1, t1, wv, lw2v, lb2v, outv,
                     sa0, sb0, st0, sa1, sb1, st1):
    c = lax.axis_index("c")
    s = lax.axis_index("s")
    w = c * NS + s
    base = w * EPW
    pltpu.sync_copy(row3d.at[w], rowv)
    pltpu.sync_copy(col3d.at[w], colv)
    pltpu.sync_copy(wrow, wv)
    pltpu.sync_copy(lw2, lw2v)
    pltpu.sync_copy(lb2, lb2v)

    def start(j, ab, bb, tb, s1_, s2_, s3_):
        pltpu.async_copy(a.at[rowv.at[j]], ab, s1_)
        pltpu.async_copy(b.at[colv.at[j]], bb, s2_)
        pltpu.async_copy(attr.at[pl.ds(base + j * C, C)], tb, s3_)

    def drain(ab, bb, tb, s1_, s2_, s3_):
        pltpu.make_async_copy(a.at[rowv.at[0]], ab, s1_).wait()
        pltpu.make_async_copy(b.at[colv.at[0]], bb, s2_).wait()
        pltpu.make_async_copy(attr.at[pl.ds(base, C)], tb, s3_).wait()

    def compute(j, ab, bb, tb):
        lb2s = lb2v[pl.ds(0, 16)][0]
        for g in range(C // 16):
            eidx = lax.iota(jnp.int32, 16) + (g * 16)
            attr_vec = tb[pl.ds(g * 16, 16)]

            def dblk(k, accv):
                wblk = wv[pl.ds(k * 16, 16)]
                lwblk = lw2v[pl.ds(k * 16, 16)]
                for dd in range(16):
                    dvec = jnp.full((16,), k * 16 + dd, jnp.int32)
                    av = plsc.load_gather(ab, [eidx, dvec])
                    bv = plsc.load_gather(bb, [eidx, dvec])
                    sv = jnp.maximum(av + bv + attr_vec * wblk[dd], 0.0)
                    accv = accv + sv * lwblk[dd]
                return accv

            acc = lax.fori_loop(0, H // 16, dblk, jnp.zeros((16,), jnp.float32))
            oidx = lax.iota(jnp.int32, 16) + (j * C + g * 16)
            plsc.store_scatter(outv, [oidx], acc + lb2s)

    start(0, a0, b0, t0, sa0, sb0, st0)

    def pair(i, _):
        j0 = 2 * i
        start(j0 + 1, a1, b1, t1, sa1, sb1, st1)
        drain(a0, b0, t0, sa0, sb0, st0)
        compute(j0, a0, b0, t0)
        start(j0 + 2, a0, b0, t0, sa0, sb0, st0)
        drain(a1, b1, t1, sa1, sb1, st1)
        compute(j0 + 1, a1, b1, t1)
        return 0

    lax.fori_loop(0, (RPW - 1) // 2, pair, 0)
    drain(a0, b0, t0, sa0, sb0, st0)
    compute(RPW - 1, a0, b0, t0)
    pltpu.sync_copy(outv, out.at[pl.ds(base, EPW)])


BM = 1000


def _tc1_body(x_ref, w1_ref, degp_ref, y1_ref, dis_ref):
    deg = degp_ref[:, 0] + degp_ref[:, 1] + 1.0
    dis = lax.rsqrt(deg)
    xw = jnp.dot(x_ref[...], w1_ref[...], preferred_element_type=jnp.float32)
    y1_ref[...] = xw * dis[:, None]
    dis_ref[...] = dis[:, None]


def _tc1(x, w1, degp_t):
    return pl.pallas_call(
        _tc1_body,
        grid=(N_NODES // BM,),
        in_specs=[
            pl.BlockSpec((BM, 128), lambda i: (i, 0)),
            pl.BlockSpec((128, H), lambda i: (0, 0)),
            pl.BlockSpec((BM, NC), lambda i: (i, 0)),
        ],
        out_specs=[
            pl.BlockSpec((BM, H), lambda i: (i, 0)),
            pl.BlockSpec((BM, 1), lambda i: (i, 0)),
        ],
        out_shape=[
            jax.ShapeDtypeStruct((NP, H), jnp.float32),
            jax.ShapeDtypeStruct((NP, 1), jnp.float32),
        ],
    )(x, w1, degp_t)


def _tc2_body(tp_ref, y_ref, dis_ref, w_ref, b_ref, out_ref):
    dis = dis_ref[...]
    h = jnp.maximum(
        (tp_ref[0] + tp_ref[1] + y_ref[...]) * dis + b_ref[...], 0.0)
    out_ref[...] = jnp.dot(
        h, w_ref[...], preferred_element_type=jnp.float32) * dis


def _tc2(tp, y, dis, w, b):
    return pl.pallas_call(
        _tc2_body,
        grid=(N_NODES // BM,),
        in_specs=[
            pl.BlockSpec((NC, BM, H), lambda i: (0, i, 0)),
            pl.BlockSpec((BM, H), lambda i: (i, 0)),
            pl.BlockSpec((BM, 1), lambda i: (i, 0)),
            pl.BlockSpec((H, H), lambda i: (0, 0)),
            pl.BlockSpec((1, H), lambda i: (0, 0)),
        ],
        out_specs=pl.BlockSpec((BM, H), lambda i: (i, 0)),
        out_shape=jax.ShapeDtypeStruct((NP, H), jnp.float32),
    )(tp, y, dis, w, b)


def _tc3_body(tp_ref, y_ref, dis_ref, b_ref, lwa_ref, lwb_ref, lb1_ref,
              a_ref, bo_ref):
    dis = dis_ref[...]
    h = jnp.maximum(
        (tp_ref[0] + tp_ref[1] + y_ref[...]) * dis + b_ref[...], 0.0)
    a_ref[...] = jnp.dot(
        h, lwa_ref[...], preferred_element_type=jnp.float32) + lb1_ref[...]
    bo_ref[...] = jnp.dot(h, lwb_ref[...], preferred_element_type=jnp.float32)


def _tc3(tp, y, dis, b, lwa, lwb, lb1):
    return pl.pallas_call(
        _tc3_body,
        grid=(N_NODES // BM,),
        in_specs=[
            pl.BlockSpec((NC, BM, H), lambda i: (0, i, 0)),
            pl.BlockSpec((BM, H), lambda i: (i, 0)),
            pl.BlockSpec((BM, 1), lambda i: (i, 0)),
            pl.BlockSpec((1, H), lambda i: (0, 0)),
            pl.BlockSpec((H, H), lambda i: (0, 0)),
            pl.BlockSpec((H, H), lambda i: (0, 0)),
            pl.BlockSpec((1, H), lambda i: (0, 0)),
        ],
        out_specs=[
            pl.BlockSpec((BM, H), lambda i: (i, 0)),
            pl.BlockSpec((BM, H), lambda i: (i, 0)),
        ],
        out_shape=[
            jax.ShapeDtypeStruct((NP, H), jnp.float32),
            jax.ShapeDtypeStruct((NP, H), jnp.float32),
        ],
    )(tp, y, dis, b, lwa, lwb, lb1)


def kernel(x, edge_index, edge_attr, W1, b1, W2, b2, LW1, Lb1, LW2, Lb2):
    row3d = edge_index[0].astype(jnp.int32).reshape(NW, RPW, C)
    col3d = edge_index[1].astype(jnp.int32).reshape(NW, RPW, C)
    ones_c = jnp.ones((C,), jnp.float32)
    zeros_n = jnp.zeros((NP,), jnp.float32)
    zeros_nh = jnp.zeros((NP, H), jnp.float32)

    degp = _deg_kernel(col3d, ones_c, zeros_n)
    degp_t = degp.reshape(NC, NP).T

    y1, dis = _tc1(x, W1, degp_t)
    t1p = _seg_kernel(y1, row3d, col3d, zeros_nh)
    y2 = _tc2(t1p, y1, dis, W2, b1.reshape(1, H))
    t2p = _seg_kernel(y2, row3d, col3d, zeros_nh)
    a_tab, b_tab = _tc3(t2p, y2, dis, b2.reshape(1, H),
                        LW1[:H], LW1[H:2 * H], Lb1.reshape(1, H))
    out = _edge_mlp_kernel(a_tab, b_tab, row3d, col3d,
                           edge_attr.reshape(N_EDGES), LW1[2 * H],
                           LW2[:, 0], jnp.broadcast_to(Lb2, (16,)))
    return out


# speedup vs baseline: 29.0164x; 2.4438x over previous
import functools

import jax
import jax.numpy as jnp
from jax import lax
from jax.experimental import pallas as pl
from jax.experimental.pallas import tpu as pltpu
from jax.experimental.pallas import tpu_sc as plsc

N_NODES = 10000
N_EDGES = 320000
NC = 2
NS = 16
NW = NC * NS
C = 80
EPW = N_EDGES // NW
RPW = EPW // C
NP = 10240
NPS = NP // NS
H = 64

_mesh = plsc.VectorSubcoreMesh(
    core_axis_name="c", subcore_axis_name="s", num_cores=NC, num_subcores=NS)
_sc_params = pltpu.CompilerParams(use_tc_tiling_on_sc=False)
_sc_params_nl = pltpu.CompilerParams(
    use_tc_tiling_on_sc=False, needs_layout_passes=False)


@functools.partial(
    pl.kernel,
    out_type=jax.ShapeDtypeStruct((NC * NP,), jnp.float32),
    mesh=_mesh,
    compiler_params=_sc_params,
    scratch_types=[
        pltpu.VMEM((RPW, C), jnp.int32),
        pltpu.VMEM((C,), jnp.float32),
        pltpu.VMEM_SHARED((NP,), jnp.float32),
    ],
)
def _deg_kernel(col3d, ones, zerosn, degp, colv, onesv, acc):
    c = lax.axis_index("c")
    s = lax.axis_index("s")
    w = c * NS + s
    pltpu.sync_copy(zerosn.at[pl.ds(s * NPS, NPS)], acc.at[pl.ds(s * NPS, NPS)])
    pltpu.sync_copy(ones, onesv)
    pltpu.sync_copy(col3d.at[w], colv)
    plsc.subcore_barrier()

    def body(j, _):
        pltpu.sync_copy(onesv, acc.at[colv.at[j]], add=True)
        return 0

    lax.fori_loop(0, RPW, body, 0)
    plsc.subcore_barrier()
    pltpu.sync_copy(acc.at[pl.ds(s * NPS, NPS)],
                    degp.at[pl.ds(c * NP + s * NPS, NPS)])


@functools.partial(
    pl.kernel,
    out_type=jax.ShapeDtypeStruct((NC, NP, H), jnp.float32),
    mesh=_mesh,
    compiler_params=_sc_params,
    scratch_types=[
        pltpu.VMEM((RPW, C), jnp.int32),
        pltpu.VMEM((RPW, C), jnp.int32),
        pltpu.VMEM((C, H), jnp.float32),
        pltpu.VMEM((C, H), jnp.float32),
        pltpu.VMEM_SHARED((NP, H), jnp.float32),
        pltpu.SemaphoreType.DMA,
        pltpu.SemaphoreType.DMA,
    ],
)
def _seg_kernel(y, row3d, col3d, zerosnh, tp, rowv, colv, g0, g1, acc, s0, s1):
    c = lax.axis_index("c")
    s = lax.axis_index("s")
    w = c * NS + s
    pltpu.sync_copy(zerosnh.at[pl.ds(s * NPS, NPS)], acc.at[pl.ds(s * NPS, NPS)])
    pltpu.sync_copy(row3d.at[w], rowv)
    pltpu.sync_copy(col3d.at[w], colv)
    plsc.subcore_barrier()

    pltpu.async_copy(y.at[rowv.at[0]], g0, s0)

    def pair(i, _):
        j0 = 2 * i
        pltpu.async_copy(y.at[rowv.at[j0 + 1]], g1, s1)
        pltpu.make_async_copy(y.at[rowv.at[j0]], g0, s0).wait()
        pltpu.sync_copy(g0, acc.at[colv.at[j0]], add=True)
        pltpu.async_copy(y.at[rowv.at[j0 + 2]], g0, s0)
        pltpu.make_async_copy(y.at[rowv.at[j0 + 1]], g1, s1).wait()
        pltpu.sync_copy(g1, acc.at[colv.at[j0 + 1]], add=True)
        return 0

    lax.fori_loop(0, (RPW - 1) // 2, pair, 0)
    pltpu.make_async_copy(y.at[rowv.at[RPW - 1]], g0, s0).wait()
    pltpu.sync_copy(g0, acc.at[colv.at[RPW - 1]], add=True)
    plsc.subcore_barrier()
    pltpu.sync_copy(acc.at[pl.ds(s * NPS, NPS)], tp.at[c, pl.ds(s * NPS, NPS)])


@functools.partial(
    pl.kernel,
    out_type=jax.ShapeDtypeStruct((N_EDGES,), jnp.float32),
    mesh=_mesh,
    compiler_params=_sc_params_nl,
    scratch_types=[
        pltpu.VMEM((RPW, C), jnp.int32),
        pltpu.VMEM((RPW, C), jnp.int32),
        pltpu.VMEM((C, H), jnp.float32),
        pltpu.VMEM((C, H), jnp.float32),
        pltpu.VMEM((C,), jnp.float32),
        pltpu.VMEM((C, H), jnp.float32),
        pltpu.VMEM((C, H), jnp.float32),
        pltpu.VMEM((C,), jnp.float32),
        pltpu.VMEM((H,), jnp.float32),
        pltpu.VMEM((H,), jnp.float32),
        pltpu.VMEM((16,), jnp.float32),
        pltpu.VMEM((EPW,), jnp.float32),
        pltpu.SemaphoreType.DMA,
        pltpu.SemaphoreType.DMA,
        pltpu.SemaphoreType.DMA,
        pltpu.SemaphoreType.DMA,
        pltpu.SemaphoreType.DMA,
        pltpu.SemaphoreType.DMA,
    ],
)
def _edge_mlp_kernel(a, b, row3d, col3d, attr, wrow, lw2, lb2, out,
                     rowv, colv, a0, b0, t0, a1, b1, t1, wv, lw2v, lb2v, outv,
                     sa0, sb0, st0, sa1, sb1, st1):
    c = lax.axis_index("c")
    s = lax.axis_index("s")
    w = c * NS + s
    base = w * EPW
    pltpu.sync_copy(row3d.at[w], rowv)
    pltpu.sync_copy(col3d.at[w], colv)
    pltpu.sync_copy(wrow, wv)
    pltpu.sync_copy(lw2, lw2v)
    pltpu.sync_copy(lb2, lb2v)

    def start(j, ab, bb, tb, s1_, s2_, s3_):
        pltpu.async_copy(a.at[rowv.at[j]], ab, s1_)
        pltpu.async_copy(b.at[colv.at[j]], bb, s2_)
        pltpu.async_copy(attr.at[pl.ds(base + j * C, C)], tb, s3_)

    def drain(ab, bb, tb, s1_, s2_, s3_):
        pltpu.make_async_copy(a.at[rowv.at[0]], ab, s1_).wait()
        pltpu.make_async_copy(b.at[colv.at[0]], bb, s2_).wait()
        pltpu.make_async_copy(attr.at[pl.ds(base, C)], tb, s3_).wait()

    lb2s = lb2v[pl.ds(0, 16)][0]
    wqs = [wv[pl.ds(q * 16, 16)] for q in range(H // 16)]
    lqs = [lw2v[pl.ds(q * 16, 16)] for q in range(H // 16)]
    lane = lax.iota(jnp.int32, 16)

    def compute(j, ab, bb, tb):
        def grp(g, _):
            attr_vec = tb[pl.ds(g * 16, 16)]
            out_acc = jnp.zeros((16,), jnp.float32)
            for i in range(16):
                e = g * 16 + i
                attr_e = attr_vec[i]
                tsum = None
                for q in range(H // 16):
                    av = ab[e, pl.ds(q * 16, 16)]
                    bv = bb[e, pl.ds(q * 16, 16)]
                    r = jnp.maximum(av + bv + attr_e * wqs[q], 0.0) * lqs[q]
                    tsum = r if tsum is None else tsum + r
                out_acc = jnp.where(lane == i, jnp.sum(tsum), out_acc)
            oidx = lane + (j * C + g * 16)
            plsc.store_scatter(outv, [oidx], out_acc + lb2s)
            return 0

        lax.fori_loop(0, C // 16, grp, 0)

    start(0, a0, b0, t0, sa0, sb0, st0)

    def pair(i, _):
        j0 = 2 * i
        start(j0 + 1, a1, b1, t1, sa1, sb1, st1)
        drain(a0, b0, t0, sa0, sb0, st0)
        compute(j0, a0, b0, t0)
        start(j0 + 2, a0, b0, t0, sa0, sb0, st0)
        drain(a1, b1, t1, sa1, sb1, st1)
        compute(j0 + 1, a1, b1, t1)
        return 0

    lax.fori_loop(0, (RPW - 1) // 2, pair, 0)
    drain(a0, b0, t0, sa0, sb0, st0)
    compute(RPW - 1, a0, b0, t0)
    pltpu.sync_copy(outv, out.at[pl.ds(base, EPW)])


BM = 1000


def _tc1_body(x_ref, w1_ref, degp_ref, y1_ref, dis_ref):
    deg = degp_ref[:, 0] + degp_ref[:, 1] + 1.0
    dis = lax.rsqrt(deg)
    xw = jnp.dot(x_ref[...], w1_ref[...], preferred_element_type=jnp.float32)
    y1_ref[...] = xw * dis[:, None]
    dis_ref[...] = dis[:, None]


def _tc1(x, w1, degp_t):
    return pl.pallas_call(
        _tc1_body,
        grid=(N_NODES // BM,),
        in_specs=[
            pl.BlockSpec((BM, 128), lambda i: (i, 0)),
            pl.BlockSpec((128, H), lambda i: (0, 0)),
            pl.BlockSpec((BM, NC), lambda i: (i, 0)),
        ],
        out_specs=[
            pl.BlockSpec((BM, H), lambda i: (i, 0)),
            pl.BlockSpec((BM, 1), lambda i: (i, 0)),
        ],
        out_shape=[
            jax.ShapeDtypeStruct((NP, H), jnp.float32),
            jax.ShapeDtypeStruct((NP, 1), jnp.float32),
        ],
    )(x, w1, degp_t)


def _tc2_body(tp_ref, y_ref, dis_ref, w_ref, b_ref, out_ref):
    dis = dis_ref[...]
    h = jnp.maximum(
        (tp_ref[0] + tp_ref[1] + y_ref[...]) * dis + b_ref[...], 0.0)
    out_ref[...] = jnp.dot(
        h, w_ref[...], preferred_element_type=jnp.float32) * dis


def _tc2(tp, y, dis, w, b):
    return pl.pallas_call(
        _tc2_body,
        grid=(N_NODES // BM,),
        in_specs=[
            pl.BlockSpec((NC, BM, H), lambda i: (0, i, 0)),
            pl.BlockSpec((BM, H), lambda i: (i, 0)),
            pl.BlockSpec((BM, 1), lambda i: (i, 0)),
            pl.BlockSpec((H, H), lambda i: (0, 0)),
            pl.BlockSpec((1, H), lambda i: (0, 0)),
        ],
        out_specs=pl.BlockSpec((BM, H), lambda i: (i, 0)),
        out_shape=jax.ShapeDtypeStruct((NP, H), jnp.float32),
    )(tp, y, dis, w, b)


def _tc3_body(tp_ref, y_ref, dis_ref, b_ref, lwa_ref, lwb_ref, lb1_ref,
              a_ref, bo_ref):
    dis = dis_ref[...]
    h = jnp.maximum(
        (tp_ref[0] + tp_ref[1] + y_ref[...]) * dis + b_ref[...], 0.0)
    a_ref[...] = jnp.dot(
        h, lwa_ref[...], preferred_element_type=jnp.float32) + lb1_ref[...]
    bo_ref[...] = jnp.dot(h, lwb_ref[...], preferred_element_type=jnp.float32)


def _tc3(tp, y, dis, b, lwa, lwb, lb1):
    return pl.pallas_call(
        _tc3_body,
        grid=(N_NODES // BM,),
        in_specs=[
            pl.BlockSpec((NC, BM, H), lambda i: (0, i, 0)),
            pl.BlockSpec((BM, H), lambda i: (i, 0)),
            pl.BlockSpec((BM, 1), lambda i: (i, 0)),
            pl.BlockSpec((1, H), lambda i: (0, 0)),
            pl.BlockSpec((H, H), lambda i: (0, 0)),
            pl.BlockSpec((H, H), lambda i: (0, 0)),
            pl.BlockSpec((1, H), lambda i: (0, 0)),
        ],
        out_specs=[
            pl.BlockSpec((BM, H), lambda i: (i, 0)),
            pl.BlockSpec((BM, H), lambda i: (i, 0)),
        ],
        out_shape=[
            jax.ShapeDtypeStruct((NP, H), jnp.float32),
            jax.ShapeDtypeStruct((NP, H), jnp.float32),
        ],
    )(tp, y, dis, b, lwa, lwb, lb1)


def kernel(x, edge_index, edge_attr, W1, b1, W2, b2, LW1, Lb1, LW2, Lb2):
    row3d = edge_index[0].astype(jnp.int32).reshape(NW, RPW, C)
    col3d = edge_index[1].astype(jnp.int32).reshape(NW, RPW, C)
    ones_c = jnp.ones((C,), jnp.float32)
    zeros_n = jnp.zeros((NP,), jnp.float32)
    zeros_nh = jnp.zeros((NP, H), jnp.float32)

    degp = _deg_kernel(col3d, ones_c, zeros_n)
    degp_t = degp.reshape(NC, NP).T

    y1, dis = _tc1(x, W1, degp_t)
    t1p = _seg_kernel(y1, row3d, col3d, zeros_nh)
    y2 = _tc2(t1p, y1, dis, W2, b1.reshape(1, H))
    t2p = _seg_kernel(y2, row3d, col3d, zeros_nh)
    a_tab, b_tab = _tc3(t2p, y2, dis, b2.reshape(1, H),
                        LW1[:H], LW1[H:2 * H], Lb1.reshape(1, H))
    out = _edge_mlp_kernel(a_tab, b_tab, row3d, col3d,
                           edge_attr.reshape(N_EDGES), LW1[2 * H],
                           LW2[:, 0], jnp.broadcast_to(Lb2, (16,)))
    return out


# speedup vs baseline: 32.5630x; 1.1222x over previous
import functools

import jax
import jax.numpy as jnp
from jax import lax
from jax.experimental import pallas as pl
from jax.experimental.pallas import tpu as pltpu
from jax.experimental.pallas import tpu_sc as plsc

N_NODES = 10000
N_EDGES = 320000
NC = 2
NS = 16
NW = NC * NS
C = 80
EPW = N_EDGES // NW
RPW = EPW // C
CS = 125
RPWS = EPW // CS
NP = 10240
NPS = NP // NS
H = 64

_mesh = plsc.VectorSubcoreMesh(
    core_axis_name="c", subcore_axis_name="s", num_cores=NC, num_subcores=NS)
_sc_params = pltpu.CompilerParams(use_tc_tiling_on_sc=False)
_sc_params_nl = pltpu.CompilerParams(
    use_tc_tiling_on_sc=False, needs_layout_passes=False)


@functools.partial(
    pl.kernel,
    out_type=jax.ShapeDtypeStruct((NC * NP,), jnp.float32),
    mesh=_mesh,
    compiler_params=_sc_params,
    scratch_types=[
        pltpu.VMEM((RPWS, CS), jnp.int32),
        pltpu.VMEM((CS,), jnp.float32),
        pltpu.VMEM_SHARED((NP,), jnp.float32),
    ],
)
def _deg_kernel(idx4, ones, zerosn, degp, colv, onesv, acc):
    c = lax.axis_index("c")
    s = lax.axis_index("s")
    w = c * NS + s
    pltpu.sync_copy(zerosn.at[pl.ds(s * NPS, NPS)], acc.at[pl.ds(s * NPS, NPS)])
    pltpu.sync_copy(ones, onesv)
    pltpu.sync_copy(idx4.at[1, w], colv)
    plsc.subcore_barrier()

    def body(j, _):
        pltpu.sync_copy(onesv, acc.at[colv.at[j]], add=True)
        return 0

    lax.fori_loop(0, RPWS, body, 0)
    plsc.subcore_barrier()
    pltpu.sync_copy(acc.at[pl.ds(s * NPS, NPS)],
                    degp.at[pl.ds(c * NP + s * NPS, NPS)])


@functools.partial(
    pl.kernel,
    out_type=jax.ShapeDtypeStruct((NC, NP, H), jnp.float32),
    mesh=_mesh,
    compiler_params=_sc_params,
    scratch_types=[
        pltpu.VMEM((RPWS, CS), jnp.int32),
        pltpu.VMEM((RPWS, CS), jnp.int32),
        pltpu.VMEM((CS, H), jnp.float32),
        pltpu.VMEM((CS, H), jnp.float32),
        pltpu.VMEM_SHARED((NP, H), jnp.float32),
        pltpu.SemaphoreType.DMA,
        pltpu.SemaphoreType.DMA,
    ],
)
def _seg_kernel(y, idx4, zerosnh, tp, rowv, colv, g0, g1, acc, s0, s1):
    c = lax.axis_index("c")
    s = lax.axis_index("s")
    w = c * NS + s
    pltpu.sync_copy(zerosnh.at[pl.ds(s * NPS, NPS)], acc.at[pl.ds(s * NPS, NPS)])
    pltpu.sync_copy(idx4.at[0, w], rowv)
    pltpu.sync_copy(idx4.at[1, w], colv)
    plsc.subcore_barrier()

    pltpu.async_copy(y.at[rowv.at[0]], g0, s0)

    def pair(i, _):
        j0 = 2 * i
        pltpu.async_copy(y.at[rowv.at[j0 + 1]], g1, s1)
        pltpu.make_async_copy(y.at[rowv.at[j0]], g0, s0).wait()
        pltpu.sync_copy(g0, acc.at[colv.at[j0]], add=True)
        pltpu.async_copy(y.at[rowv.at[j0 + 2]], g0, s0)
        pltpu.make_async_copy(y.at[rowv.at[j0 + 1]], g1, s1).wait()
        pltpu.sync_copy(g1, acc.at[colv.at[j0 + 1]], add=True)
        return 0

    lax.fori_loop(0, RPWS // 2 - 1, pair, 0)
    pltpu.async_copy(y.at[rowv.at[RPWS - 1]], g1, s1)
    pltpu.make_async_copy(y.at[rowv.at[RPWS - 2]], g0, s0).wait()
    pltpu.sync_copy(g0, acc.at[colv.at[RPWS - 2]], add=True)
    pltpu.make_async_copy(y.at[rowv.at[RPWS - 1]], g1, s1).wait()
    pltpu.sync_copy(g1, acc.at[colv.at[RPWS - 1]], add=True)
    plsc.subcore_barrier()
    pltpu.sync_copy(acc.at[pl.ds(s * NPS, NPS)], tp.at[c, pl.ds(s * NPS, NPS)])


@functools.partial(
    pl.kernel,
    out_type=jax.ShapeDtypeStruct((N_EDGES,), jnp.float32),
    mesh=_mesh,
    compiler_params=_sc_params_nl,
    scratch_types=[
        pltpu.VMEM((RPW, C), jnp.int32),
        pltpu.VMEM((RPW, C), jnp.int32),
        pltpu.VMEM((C, H), jnp.float32),
        pltpu.VMEM((C, H), jnp.float32),
        pltpu.VMEM((C,), jnp.float32),
        pltpu.VMEM((C, H), jnp.float32),
        pltpu.VMEM((C, H), jnp.float32),
        pltpu.VMEM((C,), jnp.float32),
        pltpu.VMEM((H,), jnp.float32),
        pltpu.VMEM((H,), jnp.float32),
        pltpu.VMEM((16,), jnp.float32),
        pltpu.VMEM((EPW,), jnp.float32),
        pltpu.SemaphoreType.DMA,
        pltpu.SemaphoreType.DMA,
        pltpu.SemaphoreType.DMA,
        pltpu.SemaphoreType.DMA,
        pltpu.SemaphoreType.DMA,
        pltpu.SemaphoreType.DMA,
    ],
)
def _edge_mlp_kernel(a, b, idx4, attr, wrow, lw2, lb2, out,
                     rowv, colv, a0, b0, t0, a1, b1, t1, wv, lw2v, lb2v, outv,
                     sa0, sb0, st0, sa1, sb1, st1):
    c = lax.axis_index("c")
    s = lax.axis_index("s")
    w = c * NS + s
    base = w * EPW
    pltpu.sync_copy(idx4.at[0, w], rowv)
    pltpu.sync_copy(idx4.at[1, w], colv)
    pltpu.sync_copy(wrow, wv)
    pltpu.sync_copy(lw2, lw2v)
    pltpu.sync_copy(lb2, lb2v)

    def start(j, ab, bb, tb, s1_, s2_, s3_):
        pltpu.async_copy(a.at[rowv.at[j]], ab, s1_)
        pltpu.async_copy(b.at[colv.at[j]], bb, s2_)
        pltpu.async_copy(attr.at[pl.ds(base + j * C, C)], tb, s3_)

    def drain(ab, bb, tb, s1_, s2_, s3_):
        pltpu.make_async_copy(a.at[rowv.at[0]], ab, s1_).wait()
        pltpu.make_async_copy(b.at[colv.at[0]], bb, s2_).wait()
        pltpu.make_async_copy(attr.at[pl.ds(base, C)], tb, s3_).wait()

    lb2s = lb2v[pl.ds(0, 16)][0]
    wqs = [wv[pl.ds(q * 16, 16)] for q in range(H // 16)]
    lqs = [lw2v[pl.ds(q * 16, 16)] for q in range(H // 16)]
    lane = lax.iota(jnp.int32, 16)

    def compute(j, ab, bb, tb):
        def grp(g, _):
            attr_vec = tb[pl.ds(g * 16, 16)]
            out_acc = jnp.zeros((16,), jnp.float32)
            for i in range(16):
                e = g * 16 + i
                attr_e = attr_vec[i]
                tsum = None
                for q in range(H // 16):
                    av = ab[e, pl.ds(q * 16, 16)]
                    bv = bb[e, pl.ds(q * 16, 16)]
                    r = jnp.maximum(av + bv + attr_e * wqs[q], 0.0) * lqs[q]
                    tsum = r if tsum is None else tsum + r
                out_acc = jnp.where(lane == i, jnp.sum(tsum), out_acc)
            oidx = lane + (j * C + g * 16)
            plsc.store_scatter(outv, [oidx], out_acc + lb2s)
            return 0

        lax.fori_loop(0, C // 16, grp, 0)

    start(0, a0, b0, t0, sa0, sb0, st0)

    def pair(i, _):
        j0 = 2 * i
        start(j0 + 1, a1, b1, t1, sa1, sb1, st1)
        drain(a0, b0, t0, sa0, sb0, st0)
        compute(j0, a0, b0, t0)
        start(j0 + 2, a0, b0, t0, sa0, sb0, st0)
        drain(a1, b1, t1, sa1, sb1, st1)
        compute(j0 + 1, a1, b1, t1)
        return 0

    lax.fori_loop(0, (RPW - 1) // 2, pair, 0)
    drain(a0, b0, t0, sa0, sb0, st0)
    compute(RPW - 1, a0, b0, t0)
    pltpu.sync_copy(outv, out.at[pl.ds(base, EPW)])


BM = 2000


def _tcmm_body(x_ref, w1_ref, xw_ref):
    xw_ref[...] = jnp.dot(x_ref[...], w1_ref[...],
                          preferred_element_type=jnp.float32)


def _tcmm(x, w1):
    return pl.pallas_call(
        _tcmm_body,
        grid=(N_NODES // BM,),
        in_specs=[
            pl.BlockSpec((BM, 128), lambda i: (i, 0)),
            pl.BlockSpec((128, H), lambda i: (0, 0)),
        ],
        out_specs=pl.BlockSpec((BM, H), lambda i: (i, 0)),
        out_shape=jax.ShapeDtypeStruct((NP, H), jnp.float32),
    )(x, w1)


def _tcscale_body(xw_ref, degp_ref, y1_ref, dis_ref):
    deg = degp_ref[:, 0] + degp_ref[:, 1] + 1.0
    dis = lax.rsqrt(deg)
    y1_ref[...] = xw_ref[...] * dis[:, None]
    dis_ref[...] = dis[:, None]


def _tcscale(xw, degp_t):
    return pl.pallas_call(
        _tcscale_body,
        grid=(N_NODES // BM,),
        in_specs=[
            pl.BlockSpec((BM, H), lambda i: (i, 0)),
            pl.BlockSpec((BM, NC), lambda i: (i, 0)),
        ],
        out_specs=[
            pl.BlockSpec((BM, H), lambda i: (i, 0)),
            pl.BlockSpec((BM, 1), lambda i: (i, 0)),
        ],
        out_shape=[
            jax.ShapeDtypeStruct((NP, H), jnp.float32),
            jax.ShapeDtypeStruct((NP, 1), jnp.float32),
        ],
    )(xw, degp_t)


def _tc2_body(tp_ref, y_ref, dis_ref, w_ref, b_ref, out_ref):
    dis = dis_ref[...]
    h = jnp.maximum(
        (tp_ref[0] + tp_ref[1] + y_ref[...]) * dis + b_ref[...], 0.0)
    out_ref[...] = jnp.dot(
        h, w_ref[...], preferred_element_type=jnp.float32) * dis


def _tc2(tp, y, dis, w, b):
    return pl.pallas_call(
        _tc2_body,
        grid=(N_NODES // BM,),
        in_specs=[
            pl.BlockSpec((NC, BM, H), lambda i: (0, i, 0)),
            pl.BlockSpec((BM, H), lambda i: (i, 0)),
            pl.BlockSpec((BM, 1), lambda i: (i, 0)),
            pl.BlockSpec((H, H), lambda i: (0, 0)),
            pl.BlockSpec((1, H), lambda i: (0, 0)),
        ],
        out_specs=pl.BlockSpec((BM, H), lambda i: (i, 0)),
        out_shape=jax.ShapeDtypeStruct((NP, H), jnp.float32),
    )(tp, y, dis, w, b)


def _tc3_body(tp_ref, y_ref, dis_ref, b_ref, lwa_ref, lwb_ref, lb1_ref,
              a_ref, bo_ref):
    dis = dis_ref[...]
    h = jnp.maximum(
        (tp_ref[0] + tp_ref[1] + y_ref[...]) * dis + b_ref[...], 0.0)
    a_ref[...] = jnp.dot(
        h, lwa_ref[...], preferred_element_type=jnp.float32) + lb1_ref[...]
    bo_ref[...] = jnp.dot(h, lwb_ref[...], preferred_element_type=jnp.float32)


def _tc3(tp, y, dis, b, lwa, lwb, lb1):
    return pl.pallas_call(
        _tc3_body,
        grid=(N_NODES // BM,),
        in_specs=[
            pl.BlockSpec((NC, BM, H), lambda i: (0, i, 0)),
            pl.BlockSpec((BM, H), lambda i: (i, 0)),
            pl.BlockSpec((BM, 1), lambda i: (i, 0)),
            pl.BlockSpec((1, H), lambda i: (0, 0)),
            pl.BlockSpec((H, H), lambda i: (0, 0)),
            pl.BlockSpec((H, H), lambda i: (0, 0)),
            pl.BlockSpec((1, H), lambda i: (0, 0)),
        ],
        out_specs=[
            pl.BlockSpec((BM, H), lambda i: (i, 0)),
            pl.BlockSpec((BM, H), lambda i: (i, 0)),
        ],
        out_shape=[
            jax.ShapeDtypeStruct((NP, H), jnp.float32),
            jax.ShapeDtypeStruct((NP, H), jnp.float32),
        ],
    )(tp, y, dis, b, lwa, lwb, lb1)


def kernel(x, edge_index, edge_attr, W1, b1, W2, b2, LW1, Lb1, LW2, Lb2):
    idx = edge_index.astype(jnp.int32)
    idx4s = idx.reshape(2, NW, RPWS, CS)
    idx4e = idx.reshape(2, NW, RPW, C)
    ones_c = jnp.ones((CS,), jnp.float32)
    zeros_n = jnp.zeros((NP,), jnp.float32)
    zeros_nh = jnp.zeros((NP, H), jnp.float32)

    degp = _deg_kernel(idx4s, ones_c, zeros_n)
    degp_t = degp.reshape(NC, NP).T
    xw1 = _tcmm(x, W1)
    y1, dis = _tcscale(xw1, degp_t)
    t1p = _seg_kernel(y1, idx4s, zeros_nh)
    y2 = _tc2(t1p, y1, dis, W2, b1.reshape(1, H))
    t2p = _seg_kernel(y2, idx4s, zeros_nh)
    a_tab, b_tab = _tc3(t2p, y2, dis, b2.reshape(1, H),
                        LW1[:H], LW1[H:2 * H], Lb1.reshape(1, H))
    out = _edge_mlp_kernel(a_tab, b_tab, idx4e,
                           edge_attr.reshape(N_EDGES), LW1[2 * H],
                           LW2[:, 0], jnp.broadcast_to(Lb2, (16,)))
    return out


# speedup vs baseline: 39.1337x; 1.2018x over previous
import functools

import jax
import jax.numpy as jnp
from jax import lax
from jax.experimental import pallas as pl
from jax.experimental.pallas import tpu as pltpu
from jax.experimental.pallas import tpu_sc as plsc

N_NODES = 10000
N_EDGES = 320000
NC = 2
NS = 16
NW = NC * NS
C = 80
EPW = N_EDGES // NW
RPW = EPW // C
CS = 125
RPWS = EPW // CS
NP = 10240
NPS = NP // NS
H = 64

_mesh = plsc.VectorSubcoreMesh(
    core_axis_name="c", subcore_axis_name="s", num_cores=NC, num_subcores=NS)
_sc_params = pltpu.CompilerParams(use_tc_tiling_on_sc=False)
_sc_params_nl = pltpu.CompilerParams(
    use_tc_tiling_on_sc=False, needs_layout_passes=False)


@functools.partial(
    pl.kernel,
    out_type=jax.ShapeDtypeStruct((NC * NP,), jnp.float32),
    mesh=_mesh,
    compiler_params=_sc_params,
    scratch_types=[
        pltpu.VMEM((RPWS, CS), jnp.int32),
        pltpu.VMEM((CS,), jnp.float32),
        pltpu.VMEM_SHARED((NP,), jnp.float32),
    ],
)
def _deg_kernel(idx4, ones, zerosn, degp, colv, onesv, acc):
    c = lax.axis_index("c")
    s = lax.axis_index("s")
    w = c * NS + s
    pltpu.sync_copy(zerosn.at[pl.ds(s * NPS, NPS)], acc.at[pl.ds(s * NPS, NPS)])
    pltpu.sync_copy(ones, onesv)
    pltpu.sync_copy(idx4.at[1, w], colv)
    plsc.subcore_barrier()

    def body(j, _):
        pltpu.sync_copy(onesv, acc.at[colv.at[j]], add=True)
        return 0

    lax.fori_loop(0, RPWS, body, 0)
    plsc.subcore_barrier()
    pltpu.sync_copy(acc.at[pl.ds(s * NPS, NPS)],
                    degp.at[pl.ds(c * NP + s * NPS, NPS)])


@functools.partial(
    pl.kernel,
    out_type=jax.ShapeDtypeStruct((NC, NP, H), jnp.float32),
    mesh=_mesh,
    compiler_params=_sc_params,
    scratch_types=[
        pltpu.VMEM((RPWS, CS), jnp.int32),
        pltpu.VMEM((RPWS, CS), jnp.int32),
        pltpu.VMEM((CS, H), jnp.float32),
        pltpu.VMEM((CS, H), jnp.float32),
        pltpu.VMEM((CS, H), jnp.float32),
        pltpu.VMEM((CS, H), jnp.float32),
        pltpu.VMEM_SHARED((NP, H), jnp.float32),
        pltpu.SemaphoreType.DMA,
        pltpu.SemaphoreType.DMA,
        pltpu.SemaphoreType.DMA,
        pltpu.SemaphoreType.DMA,
    ],
)
def _seg_kernel(y, idx4, zerosnh, tp, rowv, colv, g0, g1, g2, g3, acc,
                s0, s1, s2, s3):
    c = lax.axis_index("c")
    s = lax.axis_index("s")
    w = c * NS + s
    gs = [g0, g1, g2, g3]
    sems = [s0, s1, s2, s3]
    pltpu.sync_copy(zerosnh.at[pl.ds(s * NPS, NPS)], acc.at[pl.ds(s * NPS, NPS)])
    pltpu.sync_copy(idx4.at[0, w], rowv)
    pltpu.sync_copy(idx4.at[1, w], colv)
    plsc.subcore_barrier()

    for b in range(4):
        pltpu.async_copy(y.at[rowv.at[b]], gs[b], sems[b])

    def it(i, _):
        for b in range(4):
            j = 4 * i + b
            pltpu.make_async_copy(y.at[rowv.at[0]], gs[b], sems[b]).wait()
            pltpu.sync_copy(gs[b], acc.at[colv.at[j]], add=True)
            pltpu.async_copy(y.at[rowv.at[j + 4]], gs[b], sems[b])
        return 0

    lax.fori_loop(0, RPWS // 4 - 1, it, 0)
    for b in range(4):
        j = RPWS - 4 + b
        pltpu.make_async_copy(y.at[rowv.at[0]], gs[b], sems[b]).wait()
        pltpu.sync_copy(gs[b], acc.at[colv.at[j]], add=True)
    plsc.subcore_barrier()
    pltpu.sync_copy(acc.at[pl.ds(s * NPS, NPS)], tp.at[c, pl.ds(s * NPS, NPS)])


@functools.partial(
    pl.kernel,
    out_type=jax.ShapeDtypeStruct((N_EDGES,), jnp.float32),
    mesh=_mesh,
    compiler_params=_sc_params_nl,
    scratch_types=[
        pltpu.VMEM((RPW, C), jnp.int32),
        pltpu.VMEM((RPW, C), jnp.int32),
        pltpu.VMEM((C, H), jnp.float32),
        pltpu.VMEM((C, H), jnp.float32),
        pltpu.VMEM((C, H), jnp.float32),
        pltpu.VMEM((C, H), jnp.float32),
        pltpu.VMEM((C, H), jnp.float32),
        pltpu.VMEM((C, H), jnp.float32),
        pltpu.VMEM((C, H), jnp.float32),
        pltpu.VMEM((C, H), jnp.float32),
        pltpu.VMEM((EPW,), jnp.float32),
        pltpu.VMEM((H,), jnp.float32),
        pltpu.VMEM((H,), jnp.float32),
        pltpu.VMEM((16,), jnp.float32),
        pltpu.VMEM((EPW,), jnp.float32),
        pltpu.SemaphoreType.DMA,
        pltpu.SemaphoreType.DMA,
        pltpu.SemaphoreType.DMA,
        pltpu.SemaphoreType.DMA,
        pltpu.SemaphoreType.DMA,
        pltpu.SemaphoreType.DMA,
        pltpu.SemaphoreType.DMA,
        pltpu.SemaphoreType.DMA,
    ],
)
def _edge_mlp_kernel(a, b, idx4, attr, wrow, lw2, lb2, out,
                     rowv, colv, a0, a1, a2, a3, b0, b1, b2, b3,
                     attrv, wv, lw2v, lb2v, outv,
                     sa0, sa1, sa2, sa3, sb0, sb1, sb2, sb3):
    c = lax.axis_index("c")
    s = lax.axis_index("s")
    w = c * NS + s
    base = w * EPW
    avs = [a0, a1, a2, a3]
    bvs = [b0, b1, b2, b3]
    sas = [sa0, sa1, sa2, sa3]
    sbs = [sb0, sb1, sb2, sb3]
    pltpu.sync_copy(idx4.at[0, w], rowv)
    pltpu.sync_copy(idx4.at[1, w], colv)
    pltpu.sync_copy(wrow, wv)
    pltpu.sync_copy(lw2, lw2v)
    pltpu.sync_copy(lb2, lb2v)
    pltpu.sync_copy(attr.at[pl.ds(base, EPW)], attrv)

    lb2s = lb2v[pl.ds(0, 16)][0]
    wqs = [wv[pl.ds(q * 16, 16)] for q in range(H // 16)]
    lqs = [lw2v[pl.ds(q * 16, 16)] for q in range(H // 16)]
    lane = lax.iota(jnp.int32, 16)

    def start(j, sl):
        pltpu.async_copy(a.at[rowv.at[j]], avs[sl], sas[sl])
        pltpu.async_copy(b.at[colv.at[j]], bvs[sl], sbs[sl])

    def drain(sl):
        pltpu.make_async_copy(a.at[rowv.at[0]], avs[sl], sas[sl]).wait()
        pltpu.make_async_copy(b.at[colv.at[0]], bvs[sl], sbs[sl]).wait()

    def compute(j, ab, bb):
        def grp(g, _):
            attr_vec = attrv[pl.ds(j * C + g * 16, 16)]
            out_acc = jnp.zeros((16,), jnp.float32)
            for i in range(16):
                e = g * 16 + i
                attr_e = attr_vec[i]
                tsum = None
                for q in range(H // 16):
                    av = ab[e, pl.ds(q * 16, 16)]
                    bv = bb[e, pl.ds(q * 16, 16)]
                    r = jnp.maximum(av + bv + attr_e * wqs[q], 0.0) * lqs[q]
                    tsum = r if tsum is None else tsum + r
                out_acc = jnp.where(lane == i, jnp.sum(tsum), out_acc)
            oidx = lane + (j * C + g * 16)
            plsc.store_scatter(outv, [oidx], out_acc + lb2s)
            return 0

        lax.fori_loop(0, C // 16, grp, 0)

    for sl in range(4):
        start(sl, sl)

    def it(i, _):
        for sl in range(4):
            j = 4 * i + sl
            drain(sl)
            compute(j, avs[sl], bvs[sl])

            @pl.when(j + 4 < RPW)
            def _():
                start(j + 4, sl)
        return 0

    lax.fori_loop(0, (RPW - 1) // 4, it, 0)
    drain(0)
    compute(RPW - 1, avs[0], bvs[0])
    pltpu.sync_copy(outv, out.at[pl.ds(base, EPW)])


BM = 2000


def _tcmm_body(x_ref, w1_ref, xw_ref):
    xw_ref[...] = jnp.dot(x_ref[...], w1_ref[...],
                          preferred_element_type=jnp.float32)


def _tcmm(x, w1):
    return pl.pallas_call(
        _tcmm_body,
        grid=(N_NODES // BM,),
        in_specs=[
            pl.BlockSpec((BM, 128), lambda i: (i, 0)),
            pl.BlockSpec((128, H), lambda i: (0, 0)),
        ],
        out_specs=pl.BlockSpec((BM, H), lambda i: (i, 0)),
        out_shape=jax.ShapeDtypeStruct((NP, H), jnp.float32),
    )(x, w1)


def _tcscale_body(xw_ref, degp_ref, y1_ref, dis_ref):
    deg = degp_ref[:, 0] + degp_ref[:, 1] + 1.0
    dis = lax.rsqrt(deg)
    y1_ref[...] = xw_ref[...] * dis[:, None]
    dis_ref[...] = dis[:, None]


def _tcscale(xw, degp_t):
    return pl.pallas_call(
        _tcscale_body,
        grid=(N_NODES // BM,),
        in_specs=[
            pl.BlockSpec((BM, H), lambda i: (i, 0)),
            pl.BlockSpec((BM, NC), lambda i: (i, 0)),
        ],
        out_specs=[
            pl.BlockSpec((BM, H), lambda i: (i, 0)),
            pl.BlockSpec((BM, 1), lambda i: (i, 0)),
        ],
        out_shape=[
            jax.ShapeDtypeStruct((NP, H), jnp.float32),
            jax.ShapeDtypeStruct((NP, 1), jnp.float32),
        ],
    )(xw, degp_t)


def _tc2_body(tp_ref, y_ref, dis_ref, w_ref, b_ref, out_ref):
    dis = dis_ref[...]
    h = jnp.maximum(
        (tp_ref[0] + tp_ref[1] + y_ref[...]) * dis + b_ref[...], 0.0)
    out_ref[...] = jnp.dot(
        h, w_ref[...], preferred_element_type=jnp.float32) * dis


def _tc2(tp, y, dis, w, b):
    return pl.pallas_call(
        _tc2_body,
        grid=(N_NODES // BM,),
        in_specs=[
            pl.BlockSpec((NC, BM, H), lambda i: (0, i, 0)),
            pl.BlockSpec((BM, H), lambda i: (i, 0)),
            pl.BlockSpec((BM, 1), lambda i: (i, 0)),
            pl.BlockSpec((H, H), lambda i: (0, 0)),
            pl.BlockSpec((1, H), lambda i: (0, 0)),
        ],
        out_specs=pl.BlockSpec((BM, H), lambda i: (i, 0)),
        out_shape=jax.ShapeDtypeStruct((NP, H), jnp.float32),
    )(tp, y, dis, w, b)


def _tc3_body(tp_ref, y_ref, dis_ref, b_ref, lwa_ref, lwb_ref, lb1_ref,
              a_ref, bo_ref):
    dis = dis_ref[...]
    h = jnp.maximum(
        (tp_ref[0] + tp_ref[1] + y_ref[...]) * dis + b_ref[...], 0.0)
    a_ref[...] = jnp.dot(
        h, lwa_ref[...], preferred_element_type=jnp.float32) + lb1_ref[...]
    bo_ref[...] = jnp.dot(h, lwb_ref[...], preferred_element_type=jnp.float32)


def _tc3(tp, y, dis, b, lwa, lwb, lb1):
    return pl.pallas_call(
        _tc3_body,
        grid=(N_NODES // BM,),
        in_specs=[
            pl.BlockSpec((NC, BM, H), lambda i: (0, i, 0)),
            pl.BlockSpec((BM, H), lambda i: (i, 0)),
            pl.BlockSpec((BM, 1), lambda i: (i, 0)),
            pl.BlockSpec((1, H), lambda i: (0, 0)),
            pl.BlockSpec((H, H), lambda i: (0, 0)),
            pl.BlockSpec((H, H), lambda i: (0, 0)),
            pl.BlockSpec((1, H), lambda i: (0, 0)),
        ],
        out_specs=[
            pl.BlockSpec((BM, H), lambda i: (i, 0)),
            pl.BlockSpec((BM, H), lambda i: (i, 0)),
        ],
        out_shape=[
            jax.ShapeDtypeStruct((NP, H), jnp.float32),
            jax.ShapeDtypeStruct((NP, H), jnp.float32),
        ],
    )(tp, y, dis, b, lwa, lwb, lb1)


def kernel(x, edge_index, edge_attr, W1, b1, W2, b2, LW1, Lb1, LW2, Lb2):
    idx = edge_index.astype(jnp.int32)
    idx4s = idx.reshape(2, NW, RPWS, CS)
    idx4e = idx.reshape(2, NW, RPW, C)
    ones_c = jnp.ones((CS,), jnp.float32)
    zeros_n = jnp.zeros((NP,), jnp.float32)
    zeros_nh = jnp.zeros((NP, H), jnp.float32)

    degp = _deg_kernel(idx4s, ones_c, zeros_n)
    degp_t = degp.reshape(NC, NP).T
    xw1 = _tcmm(x, W1)
    y1, dis = _tcscale(xw1, degp_t)
    t1p = _seg_kernel(y1, idx4s, zeros_nh)
    y2 = _tc2(t1p, y1, dis, W2, b1.reshape(1, H))
    t2p = _seg_kernel(y2, idx4s, zeros_nh)
    a_tab, b_tab = _tc3(t2p, y2, dis, b2.reshape(1, H),
                        LW1[:H], LW1[H:2 * H], Lb1.reshape(1, H))
    out = _edge_mlp_kernel(a_tab, b_tab, idx4e,
                           edge_attr.reshape(N_EDGES), LW1[2 * H],
                           LW2[:, 0], jnp.broadcast_to(Lb2, (16,)))
    return out
